# baseline (speedup 1.0000x reference)
import functools

import jax
import jax.numpy as jnp
from jax import lax
from jax.experimental import pallas as pl
from jax.experimental.pallas import tpu as pltpu
from jax.experimental.pallas import tpu_sc as plsc

NC = 2
NS = 16
NW = NC * NS
LANES = 16

CHUNK_SEQS = 4


def _ceil16_offsets(ctx):
  offs = []
  k = 0
  while k + LANES <= ctx:
    offs.append(k)
    k += LANES
  if k < ctx:
    offs.append(ctx - LANES)
  return offs


def _sc_body(ctx, dim, n_chunks, chunk_rows, text_hbm, table_hbm, pos_hbm,
             emb_out, len_out, idx_a, idx_b, rows_a, rows_b, pos_v, len_v,
             sem_idx_a, sem_idx_b, sem_g_a, sem_g_b, sem_o_a,
             sem_o_b, sem_pos):
  wid = lax.axis_index("s") * NC + lax.axis_index("c")
  w_row0 = wid * (n_chunks * chunk_rows)

  pltpu.async_copy(pos_hbm, pos_v, sem_pos).wait()

  iota = lax.broadcasted_iota(jnp.int32, (LANES,), 0)
  tail_offs = _ceil16_offsets(ctx)

  lo = (ctx // 2 + 7) // 8 * 8
  hi = ctx - lo
  assert lo % 8 == 0 and lo <= 128 and hi <= 128

  def start_idx(c, idx_ref, sem):
    base = w_row0 + c * chunk_rows
    pltpu.async_copy(text_hbm.at[pl.ds(base, chunk_rows)], idx_ref, sem)

  def drain_idx(idx_ref, sem):
    pltpu.make_async_copy(text_hbm.at[pl.ds(0, chunk_rows)], idx_ref,
                          sem).wait()

  def start_gather(idx_ref, rows_ref, sem):
    for s in range(CHUNK_SEQS):
      b = s * ctx
      pltpu.async_copy(table_hbm.at[idx_ref.at[pl.ds(b, lo)]],
                       rows_ref.at[pl.ds(b, lo)], sem)
      pltpu.async_copy(table_hbm.at[idx_ref.at[pl.ds(b + lo, hi)]],
                       rows_ref.at[pl.ds(b + lo, hi)], sem)

  def drain_gather(rows_ref, sem):
    pltpu.make_async_copy(table_hbm.at[pl.ds(0, chunk_rows)], rows_ref,
                          sem).wait()

  def start_out(c, rows_ref, sem):
    base = w_row0 + c * chunk_rows
    pltpu.async_copy(rows_ref, emb_out.at[pl.ds(base, chunk_rows)], sem)

  def drain_out(rows_ref, sem):
    pltpu.make_async_copy(rows_ref, emb_out.at[pl.ds(0, chunk_rows)],
                          sem).wait()

  def reduce16(vec, op):
    m = vec[0]
    for k in range(1, LANES):
      m = op(m, vec[k])
    return m

  def seq_argmax(idx_ref, base):
    vmax = idx_ref[pl.ds(base + tail_offs[0], LANES)]
    for off in tail_offs[1:]:
      vmax = jnp.maximum(vmax, idx_ref[pl.ds(base + off, LANES)])
    m = reduce16(vmax, jnp.maximum)
    vpos = jnp.full((LANES,), jnp.int32(0x7FFFFFFF), dtype=jnp.int32)
    for off in tail_offs:
      v = idx_ref[pl.ds(base + off, LANES)]
      vpos = jnp.minimum(vpos, jnp.where(v == m, iota + off, 0x7FFFFFFF))
    return reduce16(vpos, jnp.minimum)

  def add_pos(rows_ref):
    def body(t, carry):
      for s in range(CHUNK_SEQS):
        for cdim in range(dim // LANES):
          sl = pl.ds(cdim * LANES, LANES)
          p = pos_v[t, sl]
          rows_ref[s * ctx + t, sl] = rows_ref[s * ctx + t, sl] + p
      return carry
    lax.fori_loop(0, ctx, body, 0, unroll=2)

  def do_chunk(c, idx_cur, rows_cur, s_idx, s_g, s_o, idx_nxt, s_idx_nxt,
               rows_prev_free, acc):
    drain_idx(idx_cur, s_idx)

    @pl.when(rows_prev_free)
    def _():
      drain_out(rows_cur, s_o)

    start_gather(idx_cur, rows_cur, s_g)

    @pl.when(c + 1 < n_chunks)
    def _():
      start_idx(c + 1, idx_nxt, s_idx_nxt)

    lane0 = (c % 4) * CHUNK_SEQS
    for s in range(CHUNK_SEQS):
      r = seq_argmax(idx_cur, s * ctx)
      acc = jnp.where(iota == lane0 + s, r, acc)

    @pl.when(c % 4 == 3)
    def _():
      len_v[pl.ds((c // 4) * LANES, LANES)] = acc

    drain_gather(rows_cur, s_g)
    add_pos(rows_cur)
    start_out(c, rows_cur, s_o)
    return acc

  start_idx(0, idx_a, sem_idx_a)

  def outer(i, acc):
    acc = do_chunk(2 * i, idx_a, rows_a, sem_idx_a, sem_g_a, sem_o_a,
                   idx_b, sem_idx_b, i >= 1, acc)
    acc = do_chunk(2 * i + 1, idx_b, rows_b, sem_idx_b, sem_g_b, sem_o_b,
                   idx_a, sem_idx_a, i >= 1, acc)
    return acc

  lax.fori_loop(0, n_chunks // 2, outer, jnp.zeros((LANES,), jnp.int32))

  drain_out(rows_a, sem_o_a)
  drain_out(rows_b, sem_o_b)

  n_seq_w = n_chunks * CHUNK_SEQS
  pltpu.sync_copy(len_v, len_out.at[pl.ds(wid * n_seq_w, n_seq_w)])


def _mask_body(ctx, o_ref):
  r = lax.broadcasted_iota(jnp.int32, (ctx, ctx), 0)
  c = lax.broadcasted_iota(jnp.int32, (ctx, ctx), 1)
  o_ref[...] = jnp.where(c > r, -jnp.inf, 0.0).astype(jnp.float32)


def kernel(text, token_embedding, pos_embed):
  b, ctx = text.shape
  _, dim = token_embedding.shape

  rows_total = b * ctx
  assert rows_total % NW == 0
  rows_per_w = rows_total // NW
  chunk_rows = CHUNK_SEQS * ctx
  assert rows_per_w % chunk_rows == 0
  n_chunks = rows_per_w // chunk_rows
  assert n_chunks % 2 == 0
  assert dim % LANES == 0

  text_flat = text.reshape(rows_total).astype(jnp.int32)
  pos2d = pos_embed.reshape(ctx, dim)

  mesh = plsc.VectorSubcoreMesh(core_axis_name="c", subcore_axis_name="s")
  sc = pl.kernel(
      functools.partial(_sc_body, ctx, dim, n_chunks, chunk_rows),
      out_type=(
          jax.ShapeDtypeStruct((rows_total, dim), jnp.float32),
          jax.ShapeDtypeStruct((b,), jnp.int32),
      ),
      mesh=mesh,
      compiler_params=pltpu.CompilerParams(use_tc_tiling_on_sc=False),
      scratch_types=[
          pltpu.VMEM((chunk_rows,), jnp.int32),
          pltpu.VMEM((chunk_rows,), jnp.int32),
          pltpu.VMEM((chunk_rows, dim), jnp.float32),
          pltpu.VMEM((chunk_rows, dim), jnp.float32),
          pltpu.VMEM((ctx, dim), jnp.float32),
          pltpu.VMEM((rows_per_w // ctx,), jnp.int32),
          pltpu.SemaphoreType.DMA,
          pltpu.SemaphoreType.DMA,
          pltpu.SemaphoreType.DMA,
          pltpu.SemaphoreType.DMA,
          pltpu.SemaphoreType.DMA,
          pltpu.SemaphoreType.DMA,
          pltpu.SemaphoreType.DMA,
      ],
  )
  emb_flat, lengths = sc(text_flat, token_embedding, pos2d)
  token_text = emb_flat.reshape(b, ctx, dim)

  mask = pl.pallas_call(
      functools.partial(_mask_body, ctx),
      out_shape=jax.ShapeDtypeStruct((ctx, ctx), jnp.float32),
  )()

  return token_text, lengths, mask
